# baseline (speedup 1.0000x reference)
import functools

import jax
import jax.numpy as jnp
from jax import lax
from jax.experimental import pallas as pl
from jax.experimental.pallas import tpu as pltpu
from jax.experimental.pallas import tpu_sc as plsc

NC = 2
NS = 16
LANES = 16
SP = 208
HALFP = SP // 2

def _sc_sum(tokens_flat, table):
    V, D = table.shape
    B = tokens_flat.shape[0] // SP
    NW = NC * NS
    BPW = B // NW

    mesh = plsc.VectorSubcoreMesh(core_axis_name="c", subcore_axis_name="s")

    NBUF = 4
    NT = 2 * BPW
    NCH = D // LANES

    @functools.partial(
        pl.kernel,
        out_type=jax.ShapeDtypeStruct((B, D), jnp.float32),
        mesh=mesh,
        scratch_types=[
            pltpu.VMEM((BPW * SP,), jnp.int32),
            pltpu.VMEM((NBUF, HALFP, D), jnp.float32),
            pltpu.VMEM((BPW, D), jnp.float32),
            pltpu.SemaphoreType.DMA((NBUF,)),
        ],
        compiler_params=pltpu.CompilerParams(use_tc_tiling_on_sc=False),
    )
    def sc_kernel(tok_hbm, table_hbm, out_hbm, idx_v, rows_v, out_v, sems):
        wid = lax.axis_index("s") * NC + lax.axis_index("c")
        base = wid * BPW
        pltpu.sync_copy(tok_hbm.at[pl.ds(base * SP, BPW * SP)], idx_v)

        def issue(t, buf):
            pltpu.async_copy(
                table_hbm.at[idx_v.at[pl.ds(t * HALFP, HALFP)]],
                rows_v.at[buf],
                sems.at[buf],
            )

        def drain(buf):
            pltpu.make_async_copy(
                table_hbm.at[idx_v.at[pl.ds(0, HALFP)]],
                rows_v.at[buf],
                sems.at[buf],
            ).wait()

        def consume(buf, chains):
            for t in range(HALFP):
                p = (t & 1) * NCH
                for k in range(NCH):
                    chains[p + k] = (chains[p + k]
                                     + rows_v[buf, t, pl.ds(LANES * k, LANES)])
            return chains

        def flush(r, chains):
            for k in range(NCH):
                out_v[r, pl.ds(LANES * k, LANES)] = chains[k] + chains[NCH + k]

        for b in range(NBUF - 1):
            issue(b, b)

        zeros = lambda: [jnp.zeros((LANES,), jnp.float32)
                         for _ in range(2 * NCH)]

        def quad_body(i, carry):
            t0 = 4 * i
            issue(t0 + 3, 3)
            drain(0)
            ch = consume(0, zeros())
            @pl.when(t0 + 4 < NT)
            def _():
                issue(t0 + 4, 0)
            drain(1)
            flush(2 * i, consume(1, ch))
            @pl.when(t0 + 5 < NT)
            def _():
                issue(t0 + 5, 1)
            drain(2)
            ch2 = consume(2, zeros())
            @pl.when(t0 + 6 < NT)
            def _():
                issue(t0 + 6, 2)
            drain(3)
            flush(2 * i + 1, consume(3, ch2))
            return carry

        lax.fori_loop(0, NT // 4, quad_body, 0)
        pltpu.sync_copy(out_v, out_hbm.at[pl.ds(base, BPW)])

    return sc_kernel(tokens_flat, table)


def _tc_mlp(sums, tokens, row0, W1, b1, W2, b2, W3, b3, W4, b4):
    B, D = sums.shape
    S = tokens.shape[1]
    H = W1.shape[1]
    O = W2.shape[1]
    BLK = 1024

    def body(sums_ref, tok_ref, row0_ref, W1r, b1r, W2r, b2r, W3r, b3r,
             W4r, b4r, loc_ref, scale_ref):
        npad = jnp.sum((tok_ref[...] == 0).astype(jnp.float32), axis=1,
                       keepdims=True)
        a = ((sums_ref[...] - (npad + jnp.float32(SP - S)) * row0_ref[...])
             / (jnp.float32(S) - npad))
        h1 = jnp.maximum(
            jnp.dot(a, W1r[...], preferred_element_type=jnp.float32)
            + b1r[...], 0.0)
        loc_ref[...] = (
            jnp.dot(h1, W2r[...], preferred_element_type=jnp.float32)
            + b2r[...])
        h2 = jnp.maximum(
            jnp.dot(a, W3r[...], preferred_element_type=jnp.float32)
            + b3r[...], 0.0)
        scale_ref[...] = jnp.exp(
            jnp.dot(h2, W4r[...], preferred_element_type=jnp.float32)
            + b4r[...])

    full = lambda shape: pl.BlockSpec(shape, lambda i: (0, 0))
    return pl.pallas_call(
        body,
        grid=(B // BLK,),
        in_specs=[
            pl.BlockSpec((BLK, D), lambda i: (i, 0)),
            pl.BlockSpec((BLK, S), lambda i: (i, 0)),
            full((1, D)),
            full((D, H)), full((1, H)),
            full((H, O)), full((1, O)),
            full((D, H)), full((1, H)),
            full((H, O)), full((1, O)),
        ],
        out_specs=[
            pl.BlockSpec((BLK, O), lambda i: (i, 0)),
            pl.BlockSpec((BLK, O), lambda i: (i, 0)),
        ],
        out_shape=[
            jax.ShapeDtypeStruct((B, O), jnp.float32),
            jax.ShapeDtypeStruct((B, O), jnp.float32),
        ],
    )(sums, tokens, row0, W1, b1, W2, b2, W3, b3, W4, b4)


def kernel(tokens, table, W1, b1, W2, b2, W3, b3, W4, b4):
    B, S = tokens.shape
    tokens_p = jnp.concatenate(
        [tokens, jnp.zeros((B, SP - S), jnp.int32)], axis=1).reshape(-1)
    sums = _sc_sum(tokens_p, table)
    loc, scale = _tc_mlp(
        sums, tokens, table[0:1, :], W1, b1.reshape(1, -1),
        W2, b2.reshape(1, -1), W3, b3.reshape(1, -1), W4, b4.reshape(1, -1))
    return (loc, scale)


# speedup vs baseline: 1.0016x; 1.0016x over previous
import functools

import jax
import jax.numpy as jnp
from jax import lax
from jax.experimental import pallas as pl
from jax.experimental.pallas import tpu as pltpu
from jax.experimental.pallas import tpu_sc as plsc

NC = 2
NS = 16
LANES = 16
SP = 208
HALFP = SP // 2

def _sc_sum(tokens_flat, table):
    V, D = table.shape
    B = tokens_flat.shape[0] // SP
    NW = NC * NS
    BPW = B // NW

    mesh = plsc.VectorSubcoreMesh(core_axis_name="c", subcore_axis_name="s")

    @functools.partial(
        pl.kernel,
        out_type=jax.ShapeDtypeStruct((B, D), jnp.float32),
        mesh=mesh,
        scratch_types=[
            pltpu.VMEM((BPW * SP,), jnp.int32),
            pltpu.VMEM((2, SP, D), jnp.float32),
            pltpu.VMEM((BPW, D), jnp.float32),
            pltpu.SemaphoreType.DMA((2,)),
        ],
        compiler_params=pltpu.CompilerParams(use_tc_tiling_on_sc=False),
    )
    def sc_kernel(tok_hbm, table_hbm, out_hbm, idx_v, rows_v, out_v, sems):
        wid = lax.axis_index("s") * NC + lax.axis_index("c")
        base = wid * BPW
        pltpu.sync_copy(tok_hbm.at[pl.ds(base * SP, BPW * SP)], idx_v)

        def issue(r, buf):
            for j in range(2):
                pltpu.async_copy(
                    table_hbm.at[idx_v.at[pl.ds(r * SP + j * HALFP, HALFP)]],
                    rows_v.at[buf, pl.ds(j * HALFP, HALFP)],
                    sems.at[buf],
                )

        def drain(buf):
            for j in range(2):
                pltpu.make_async_copy(
                    table_hbm.at[idx_v.at[pl.ds(j * HALFP, HALFP)]],
                    rows_v.at[buf, pl.ds(j * HALFP, HALFP)],
                    sems.at[buf],
                ).wait()

        issue(0, 0)

        def pair_body(i, carry):
            for b in range(2):
                r = 2 * i + b

                @pl.when(r + 1 < BPW)
                def _():
                    issue(r + 1, 1 - b)

                drain(b)

                accs = [jnp.zeros((LANES,), jnp.float32)
                        for _ in range(2 * (D // LANES))]
                for t in range(SP):
                    p = (t & 1) * (D // LANES)
                    for k in range(D // LANES):
                        accs[p + k] = (accs[p + k]
                                       + rows_v[b, t, pl.ds(LANES * k, LANES)])
                for k in range(D // LANES):
                    out_v[r, pl.ds(LANES * k, LANES)] = (
                        accs[k] + accs[(D // LANES) + k])
            return carry

        lax.fori_loop(0, BPW // 2, pair_body, 0)
        pltpu.sync_copy(out_v, out_hbm.at[pl.ds(base, BPW)])

    return sc_kernel(tokens_flat, table)


def _tc_mlp(sums, tokens, row0, W1, b1, W2, b2, W3, b3, W4, b4):
    B, D = sums.shape
    S = tokens.shape[1]
    H = W1.shape[1]
    O = W2.shape[1]
    BLK = 1024

    def body(sums_ref, tok_ref, row0_ref, W1r, b1r, W2r, b2r, W3r, b3r,
             W4r, b4r, loc_ref, scale_ref):
        npad = jnp.sum((tok_ref[...] == 0).astype(jnp.float32), axis=1,
                       keepdims=True)
        a = ((sums_ref[...] - (npad + jnp.float32(SP - S)) * row0_ref[...])
             / (jnp.float32(S) - npad))
        h1 = jnp.maximum(
            jnp.dot(a, W1r[...], preferred_element_type=jnp.float32)
            + b1r[...], 0.0)
        loc_ref[...] = (
            jnp.dot(h1, W2r[...], preferred_element_type=jnp.float32)
            + b2r[...])
        h2 = jnp.maximum(
            jnp.dot(a, W3r[...], preferred_element_type=jnp.float32)
            + b3r[...], 0.0)
        scale_ref[...] = jnp.exp(
            jnp.dot(h2, W4r[...], preferred_element_type=jnp.float32)
            + b4r[...])

    full = lambda shape: pl.BlockSpec(shape, lambda i: (0, 0))
    return pl.pallas_call(
        body,
        grid=(B // BLK,),
        in_specs=[
            pl.BlockSpec((BLK, D), lambda i: (i, 0)),
            pl.BlockSpec((BLK, S), lambda i: (i, 0)),
            full((1, D)),
            full((D, H)), full((1, H)),
            full((H, O)), full((1, O)),
            full((D, H)), full((1, H)),
            full((H, O)), full((1, O)),
        ],
        out_specs=[
            pl.BlockSpec((BLK, O), lambda i: (i, 0)),
            pl.BlockSpec((BLK, O), lambda i: (i, 0)),
        ],
        out_shape=[
            jax.ShapeDtypeStruct((B, O), jnp.float32),
            jax.ShapeDtypeStruct((B, O), jnp.float32),
        ],
    )(sums, tokens, row0, W1, b1, W2, b2, W3, b3, W4, b4)


def kernel(tokens, table, W1, b1, W2, b2, W3, b3, W4, b4):
    B, S = tokens.shape
    tokens_p = jnp.concatenate(
        [tokens, jnp.zeros((B, SP - S), jnp.int32)], axis=1).reshape(-1)
    sums = _sc_sum(tokens_p, table)
    loc, scale = _tc_mlp(
        sums, tokens, table[0:1, :], W1, b1.reshape(1, -1),
        W2, b2.reshape(1, -1), W3, b3.reshape(1, -1), W4, b4.reshape(1, -1))
    return (loc, scale)


# speedup vs baseline: 1.0063x; 1.0046x over previous
import functools

import jax
import jax.numpy as jnp
from jax import lax
from jax.experimental import pallas as pl
from jax.experimental.pallas import tpu as pltpu
from jax.experimental.pallas import tpu_sc as plsc

NC = 2
NS = 16
LANES = 16
SP = 208
HALFP = SP // 2

def _sc_sum(tokens_flat, table):
    V, D = table.shape
    B = tokens_flat.shape[0] // SP
    NW = NC * NS
    BPW = B // NW

    mesh = plsc.VectorSubcoreMesh(core_axis_name="c", subcore_axis_name="s")

    @functools.partial(
        pl.kernel,
        out_type=jax.ShapeDtypeStruct((B, D), jnp.float32),
        mesh=mesh,
        scratch_types=[
            pltpu.VMEM((BPW * SP,), jnp.int32),
            pltpu.VMEM((2, SP, D), jnp.float32),
            pltpu.VMEM((BPW, D), jnp.float32),
            pltpu.SemaphoreType.DMA((2,)),
        ],
        compiler_params=pltpu.CompilerParams(use_tc_tiling_on_sc=False),
    )
    def sc_kernel(tok_hbm, table_hbm, out_hbm, idx_v, rows_v, out_v, sems):
        wid = lax.axis_index("s") * NC + lax.axis_index("c")
        base = wid * BPW
        pltpu.sync_copy(tok_hbm.at[pl.ds(base * SP, BPW * SP)], idx_v)

        def issue(r, buf):
            for j in range(2):
                pltpu.async_copy(
                    table_hbm.at[idx_v.at[pl.ds(r * SP + j * HALFP, HALFP)]],
                    rows_v.at[buf, pl.ds(j * HALFP, HALFP)],
                    sems.at[buf],
                )

        def drain(buf):
            for j in range(2):
                pltpu.make_async_copy(
                    table_hbm.at[idx_v.at[pl.ds(j * HALFP, HALFP)]],
                    rows_v.at[buf, pl.ds(j * HALFP, HALFP)],
                    sems.at[buf],
                ).wait()

        issue(0, 0)

        def pair_body(i, carry):
            for b in range(2):
                r = 2 * i + b

                @pl.when(r + 1 < BPW)
                def _():
                    issue(r + 1, 1 - b)

                drain(b)

                def tok_body(t, accs):
                    return [accs[k] + rows_v[b, t, pl.ds(LANES * k, LANES)]
                            for k in range(D // LANES)]

                accs = lax.fori_loop(
                    0, SP, tok_body,
                    [jnp.zeros((LANES,), jnp.float32)
                     for _ in range(D // LANES)])
                for k in range(D // LANES):
                    out_v[r, pl.ds(LANES * k, LANES)] = accs[k]
            return carry

        lax.fori_loop(0, BPW // 2, pair_body, 0)
        pltpu.sync_copy(out_v, out_hbm.at[pl.ds(base, BPW)])

    return sc_kernel(tokens_flat, table)


def _tc_mlp(sums, tokens, row0, W1, b1, W2, b2, W3, b3, W4, b4):
    B, D = sums.shape
    S = tokens.shape[1]
    H = W1.shape[1]
    O = W2.shape[1]
    BLK = 1024

    def body(sums_ref, tok_ref, row0_ref, W1r, b1r, W2r, b2r, W3r, b3r,
             W4r, b4r, loc_ref, scale_ref):
        npad = jnp.sum((tok_ref[...] == 0).astype(jnp.float32), axis=1,
                       keepdims=True)
        a = ((sums_ref[...] - (npad + jnp.float32(SP - S)) * row0_ref[...])
             / (jnp.float32(S) - npad))
        h1 = jnp.maximum(
            jnp.dot(a, W1r[...], preferred_element_type=jnp.float32)
            + b1r[...], 0.0)
        loc_ref[...] = (
            jnp.dot(h1, W2r[...], preferred_element_type=jnp.float32)
            + b2r[...])
        h2 = jnp.maximum(
            jnp.dot(a, W3r[...], preferred_element_type=jnp.float32)
            + b3r[...], 0.0)
        scale_ref[...] = jnp.exp(
            jnp.dot(h2, W4r[...], preferred_element_type=jnp.float32)
            + b4r[...])

    full = lambda shape: pl.BlockSpec(shape, lambda i: (0, 0))
    return pl.pallas_call(
        body,
        grid=(B // BLK,),
        in_specs=[
            pl.BlockSpec((BLK, D), lambda i: (i, 0)),
            pl.BlockSpec((BLK, S), lambda i: (i, 0)),
            full((1, D)),
            full((D, H)), full((1, H)),
            full((H, O)), full((1, O)),
            full((D, H)), full((1, H)),
            full((H, O)), full((1, O)),
        ],
        out_specs=[
            pl.BlockSpec((BLK, O), lambda i: (i, 0)),
            pl.BlockSpec((BLK, O), lambda i: (i, 0)),
        ],
        out_shape=[
            jax.ShapeDtypeStruct((B, O), jnp.float32),
            jax.ShapeDtypeStruct((B, O), jnp.float32),
        ],
    )(sums, tokens, row0, W1, b1, W2, b2, W3, b3, W4, b4)


def kernel(tokens, table, W1, b1, W2, b2, W3, b3, W4, b4):
    B, S = tokens.shape
    tokens_p = jnp.concatenate(
        [tokens, jnp.zeros((B, SP - S), jnp.int32)], axis=1).reshape(-1)
    sums = _sc_sum(tokens_p, table)
    loc, scale = _tc_mlp(
        sums, tokens, table[0:1, :], W1, b1.reshape(1, -1),
        W2, b2.reshape(1, -1), W3, b3.reshape(1, -1), W4, b4.reshape(1, -1))
    return (loc, scale)


# speedup vs baseline: 3.7988x; 3.7751x over previous
import functools

import jax
import jax.numpy as jnp
from jax import lax
from jax.experimental import pallas as pl
from jax.experimental.pallas import tpu as pltpu
from jax.experimental.pallas import tpu_sc as plsc

NC = 2
NS = 16
LANES = 16


def _sc_sum(tokens_flat, table, SP):
    V, D = table.shape
    B = tokens_flat.shape[0] // SP
    NW = NC * NS
    BPW = B // NW
    H0 = min(128, ((SP // 2 + 7) // 8) * 8)
    SPLITS = ((0, H0), (H0, SP - H0))
    NCH = D // LANES

    mesh = plsc.VectorSubcoreMesh(core_axis_name="c", subcore_axis_name="s")

    @functools.partial(
        pl.kernel,
        out_type=jax.ShapeDtypeStruct((B, D), jnp.float32),
        mesh=mesh,
        scratch_types=[
            pltpu.VMEM((BPW * SP,), jnp.int32),
            pltpu.VMEM((2, SP, D), jnp.float32),
            pltpu.VMEM((BPW, D), jnp.float32),
            pltpu.SemaphoreType.DMA((2,)),
        ],
        compiler_params=pltpu.CompilerParams(use_tc_tiling_on_sc=False),
    )
    def sc_kernel(tok_hbm, table_hbm, out_hbm, idx_v, rows_v, out_v, sems):
        wid = lax.axis_index("s") * NC + lax.axis_index("c")
        base = wid * BPW
        pltpu.sync_copy(tok_hbm.at[pl.ds(base * SP, BPW * SP)], idx_v)

        def issue(r, buf):
            for off, ln in SPLITS:
                pltpu.async_copy(
                    table_hbm.at[idx_v.at[pl.ds(r * SP + off, ln)]],
                    rows_v.at[buf, pl.ds(off, ln)],
                    sems.at[buf],
                )

        def drain(buf):
            for off, ln in SPLITS:
                pltpu.make_async_copy(
                    table_hbm.at[idx_v.at[pl.ds(off, ln)]],
                    rows_v.at[buf, pl.ds(off, ln)],
                    sems.at[buf],
                ).wait()

        def accum(r, buf):
            def tok_body(t, accs):
                return [accs[k] + rows_v[buf, t, pl.ds(LANES * k, LANES)]
                        for k in range(NCH)]

            accs = lax.fori_loop(
                0, SP, tok_body,
                [jnp.zeros((LANES,), jnp.float32) for _ in range(NCH)])
            for k in range(NCH):
                out_v[r, pl.ds(LANES * k, LANES)] = accs[k]

        issue(0, 0)

        def pair_body(i, carry):
            r = 2 * i
            issue(r + 1, 1)
            drain(0)
            accum(r, 0)
            issue(r + 2, 0)
            drain(1)
            accum(r + 1, 1)
            return carry

        lax.fori_loop(0, BPW // 2 - 1, pair_body, 0)
        issue(BPW - 1, 1)
        drain(0)
        accum(BPW - 2, 0)
        drain(1)
        accum(BPW - 1, 1)

        pltpu.sync_copy(out_v, out_hbm.at[pl.ds(base, BPW)])

    return sc_kernel(tokens_flat, table)


def _tc_mlp(sums, tokens, row0, W1, b1, W2, b2, W3, b3, W4, b4):
    B, D = sums.shape
    S = tokens.shape[1]
    H = W1.shape[1]
    O = W2.shape[1]
    BLK = 1024

    def body(sums_ref, tok_ref, row0_ref, W1r, b1r, W2r, b2r, W3r, b3r,
             W4r, b4r, loc_ref, scale_ref):
        npad = jnp.sum((tok_ref[...] == 0).astype(jnp.float32), axis=1,
                       keepdims=True)
        a = ((sums_ref[...] - npad * row0_ref[...])
             / (jnp.float32(S) - npad))
        h1 = jnp.maximum(
            jnp.dot(a, W1r[...], preferred_element_type=jnp.float32)
            + b1r[...], 0.0)
        loc_ref[...] = (
            jnp.dot(h1, W2r[...], preferred_element_type=jnp.float32)
            + b2r[...])
        h2 = jnp.maximum(
            jnp.dot(a, W3r[...], preferred_element_type=jnp.float32)
            + b3r[...], 0.0)
        scale_ref[...] = jnp.exp(
            jnp.dot(h2, W4r[...], preferred_element_type=jnp.float32)
            + b4r[...])

    full = lambda shape: pl.BlockSpec(shape, lambda i: (0, 0))
    return pl.pallas_call(
        body,
        grid=(B // BLK,),
        in_specs=[
            pl.BlockSpec((BLK, D), lambda i: (i, 0)),
            pl.BlockSpec((BLK, S), lambda i: (i, 0)),
            full((1, D)),
            full((D, H)), full((1, H)),
            full((H, O)), full((1, O)),
            full((D, H)), full((1, H)),
            full((H, O)), full((1, O)),
        ],
        out_specs=[
            pl.BlockSpec((BLK, O), lambda i: (i, 0)),
            pl.BlockSpec((BLK, O), lambda i: (i, 0)),
        ],
        out_shape=[
            jax.ShapeDtypeStruct((B, O), jnp.float32),
            jax.ShapeDtypeStruct((B, O), jnp.float32),
        ],
    )(sums, tokens, row0, W1, b1, W2, b2, W3, b3, W4, b4)


def kernel(tokens, table, W1, b1, W2, b2, W3, b3, W4, b4):
    B, S = tokens.shape
    sums = _sc_sum(tokens.reshape(-1), table, S)
    loc, scale = _tc_mlp(
        sums, tokens, table[0:1, :], W1, b1.reshape(1, -1),
        W2, b2.reshape(1, -1), W3, b3.reshape(1, -1), W4, b4.reshape(1, -1))
    return (loc, scale)
